# Initial kernel scaffold; baseline (speedup 1.0000x reference)
import jax
import jax.numpy as jnp
from jax.experimental import pallas as pl


def kernel(x, edge_index, W, att_src, att_dst, bias):
    raise NotImplementedError("write your pallas kernel here")


# speedup vs baseline: 383.0851x; 383.0851x over previous
import functools

import jax
import jax.numpy as jnp
from jax import lax
from jax.experimental import pallas as pl
from jax.experimental.pallas import tpu as pltpu
from jax.experimental.pallas import tpu_sc as plsc

N = 100000
E = 6400000
ROW = 128
NROWS = E // ROW
NC, NS, L = 2, 16, 16
NW = NC * NS
SCROWS = 16
ZCH = 4000
NZ = N // ZCH
NEG_SLOPE = 0.2
CLAMP = -75.0


def _edge_body(x_hbm, src_hbm, dst_hbm, w_hbm, a_hbm, d_hbm,
               ex_hbm, exh_hbm, m_hbm,
               hx, srcb, dstb, exsb, exhb, zbuf, wb, ab, db, mb,
               acc_ex, acc_exh, sem):
  cid = lax.axis_index("c")
  sid = lax.axis_index("s")
  wid = sid * NC + cid

  pltpu.sync_copy(w_hbm, wb)
  pltpu.sync_copy(a_hbm, ab)
  pltpu.sync_copy(d_hbm, db)
  pltpu.sync_copy(x_hbm, hx)
  wv = wb[...]
  av = ab[...]
  dv = db[...]

  ninf = jnp.full((L,), -jnp.inf, jnp.float32)

  def hbody(j, carry):
    mas, mad = carry
    sl = pl.ds(j * L, L)
    v = hx[sl] * wv
    hx[sl] = v
    return jnp.maximum(mas, v * av), jnp.maximum(mad, v * dv)

  mas, mad = lax.fori_loop(0, N // L, hbody, (ninf, ninf))
  s_as = mas[0]
  s_ad = mad[0]
  for i in range(1, L):
    s_as = jnp.maximum(s_as, mas[i])
    s_ad = jnp.maximum(s_ad, mad[i])
  msum = s_as + s_ad
  mglob = jnp.maximum(msum, NEG_SLOPE * msum)
  mv = jnp.full((L,), mglob, jnp.float32)

  zv = jnp.zeros((L,), jnp.float32)

  def zbody(j, _):
    zbuf[pl.ds(j * L, L)] = zv
    return 0

  lax.fori_loop(0, ZCH // L, zbody, 0)
  pltpu.sync_copy(zbuf, acc_ex.at[pl.ds(sid * ZCH, ZCH)])
  pltpu.sync_copy(zbuf, acc_exh.at[pl.ds(sid * ZCH, ZCH)])

  @pl.when(sid < NZ - NS)
  def _():
    pltpu.sync_copy(zbuf, acc_ex.at[pl.ds((sid + NS) * ZCH, ZCH)])
    pltpu.sync_copy(zbuf, acc_exh.at[pl.ds((sid + NS) * ZCH, ZCH)])

  plsc.subcore_barrier()

  base = 1560 * wid + 8 * jnp.minimum(wid, 10)
  nfull = jnp.where(wid < 10, 98, 97)
  nrem8 = jnp.where(wid < 10, 0, 1)

  def compute_rows(nrows, r0):
    def rbody(j, _):
      for k in range(ROW // L):
        sl = pl.ds(k * L, L)
        sv = srcb[j, sl]
        dvec = dstb[j, sl]
        hs = plsc.load_gather(hx, [sv])
        hd = plsc.load_gather(hx, [dvec])
        e = av * hs + dv * hd
        e = jnp.maximum(e, NEG_SLOPE * e)
        ex = jnp.exp(jnp.maximum(e - mv, CLAMP))
        exsb[j, sl] = ex
        exhb[j, sl] = ex * hs
      return 0

    lax.fori_loop(0, nrows, rbody, 0)
    del r0

  def scbody(s, _):
    r0 = base + s * SCROWS
    pltpu.sync_copy(src_hbm.at[pl.ds(r0, SCROWS)], srcb)
    pltpu.sync_copy(dst_hbm.at[pl.ds(r0, SCROWS)], dstb)
    compute_rows(SCROWS, r0)
    cps = []
    for j in range(SCROWS):
      cps.append(pltpu.async_copy(
          exsb.at[j], acc_ex.at[dstb.at[j]], sem, add=True))
      cps.append(pltpu.async_copy(
          exhb.at[j], acc_exh.at[dstb.at[j]], sem, add=True))
    for cp in cps:
      cp.wait()
    return 0

  lax.fori_loop(0, nfull, scbody, 0)

  def rembody(r, _):
    r0 = base + nfull * SCROWS + r * 8
    pltpu.sync_copy(src_hbm.at[pl.ds(r0, 8)], srcb.at[pl.ds(0, 8)])
    pltpu.sync_copy(dst_hbm.at[pl.ds(r0, 8)], dstb.at[pl.ds(0, 8)])
    compute_rows(8, r0)
    cps = []
    for j in range(8):
      cps.append(pltpu.async_copy(
          exsb.at[j], acc_ex.at[dstb.at[j]], sem, add=True))
      cps.append(pltpu.async_copy(
          exhb.at[j], acc_exh.at[dstb.at[j]], sem, add=True))
    for cp in cps:
      cp.wait()
    return 0

  lax.fori_loop(0, nrem8, rembody, 0)

  plsc.subcore_barrier()

  def writeback(off):
    pltpu.sync_copy(acc_ex.at[pl.ds(off, ZCH)], zbuf)
    pltpu.sync_copy(zbuf, ex_hbm.at[pl.ds(cid * N + off, ZCH)])
    pltpu.sync_copy(acc_exh.at[pl.ds(off, ZCH)], zbuf)
    pltpu.sync_copy(zbuf, exh_hbm.at[pl.ds(cid * N + off, ZCH)])

  writeback(sid * ZCH)

  @pl.when(sid < NZ - NS)
  def _():
    writeback((sid + NS) * ZCH)

  @pl.when(jnp.logical_and(cid == 0, sid == 0))
  def _():
    mb[...] = mv
    pltpu.sync_copy(mb, m_hbm)


def _node_body(x_hbm, ex_hbm, exh_hbm, m_hbm, w_hbm, a_hbm, d_hbm, b_hbm,
               out_hbm,
               xb, e0b, e1b, h0b, h1b, ob, wb, ab, db, bb, mb):
  cid = lax.axis_index("c")
  sid = lax.axis_index("s")
  wid = sid * NC + cid

  pltpu.sync_copy(w_hbm, wb)
  pltpu.sync_copy(a_hbm, ab)
  pltpu.sync_copy(d_hbm, db)
  pltpu.sync_copy(b_hbm, bb)
  pltpu.sync_copy(m_hbm, mb)

  @pl.when(wid < NZ)
  def _():
    off = wid * ZCH
    sl_h = pl.ds(off, ZCH)
    pltpu.sync_copy(x_hbm.at[sl_h], xb)
    pltpu.sync_copy(ex_hbm.at[pl.ds(off, ZCH)], e0b)
    pltpu.sync_copy(ex_hbm.at[pl.ds(N + off, ZCH)], e1b)
    pltpu.sync_copy(exh_hbm.at[pl.ds(off, ZCH)], h0b)
    pltpu.sync_copy(exh_hbm.at[pl.ds(N + off, ZCH)], h1b)
    wv = wb[...]
    av = ab[...]
    dv = db[...]
    bv = bb[...]
    mv = mb[...]

    def nbody(j, _):
      sl = pl.ds(j * L, L)
      h = xb[sl] * wv
      t = (av + dv) * h
      t = jnp.maximum(t, NEG_SLOPE * t)
      es = jnp.exp(jnp.maximum(t - mv, CLAMP))
      num = h0b[sl] + h1b[sl] + es * h
      den = e0b[sl] + e1b[sl] + es
      ob[sl] = num / den + bv
      return 0

    lax.fori_loop(0, ZCH // L, nbody, 0)
    pltpu.sync_copy(ob, out_hbm.at[sl_h])


def _gat_impl(x, edge_index, W, att_src, att_dst, bias):
  src = edge_index[0].astype(jnp.int32).reshape(NROWS, ROW)
  dst = edge_index[1].astype(jnp.int32).reshape(NROWS, ROW)
  xf = x.reshape(N).astype(jnp.float32)
  wv = jnp.broadcast_to(W.reshape(()).astype(jnp.float32), (L,))
  av = jnp.broadcast_to(att_src.reshape(()).astype(jnp.float32), (L,))
  dv = jnp.broadcast_to(att_dst.reshape(()).astype(jnp.float32), (L,))
  bv = jnp.broadcast_to(bias.reshape(()).astype(jnp.float32), (L,))

  mesh = plsc.VectorSubcoreMesh(
      core_axis_name="c", subcore_axis_name="s",
      num_cores=NC, num_subcores=NS)

  cparams = pltpu.CompilerParams(needs_layout_passes=False)
  edge_kernel = pl.kernel(
      _edge_body,
      compiler_params=cparams,
      out_type=(
          jax.ShapeDtypeStruct((NC * N,), jnp.float32),
          jax.ShapeDtypeStruct((NC * N,), jnp.float32),
          jax.ShapeDtypeStruct((L,), jnp.float32),
      ),
      mesh=mesh,
      scratch_types=[
          pltpu.VMEM((N,), jnp.float32),
          pltpu.VMEM((SCROWS, ROW), jnp.int32),
          pltpu.VMEM((SCROWS, ROW), jnp.int32),
          pltpu.VMEM((SCROWS, ROW), jnp.float32),
          pltpu.VMEM((SCROWS, ROW), jnp.float32),
          pltpu.VMEM((ZCH,), jnp.float32),
          pltpu.VMEM((L,), jnp.float32),
          pltpu.VMEM((L,), jnp.float32),
          pltpu.VMEM((L,), jnp.float32),
          pltpu.VMEM((L,), jnp.float32),
          pltpu.VMEM_SHARED((N,), jnp.float32),
          pltpu.VMEM_SHARED((N,), jnp.float32),
          pltpu.SemaphoreType.DMA,
      ],
  )
  ex_acc, exh_acc, mvec = edge_kernel(xf, src, dst, wv, av, dv)

  node_kernel = pl.kernel(
      _node_body,
      compiler_params=cparams,
      out_type=jax.ShapeDtypeStruct((N,), jnp.float32),
      mesh=mesh,
      scratch_types=[
          pltpu.VMEM((ZCH,), jnp.float32),
          pltpu.VMEM((ZCH,), jnp.float32),
          pltpu.VMEM((ZCH,), jnp.float32),
          pltpu.VMEM((ZCH,), jnp.float32),
          pltpu.VMEM((ZCH,), jnp.float32),
          pltpu.VMEM((ZCH,), jnp.float32),
          pltpu.VMEM((L,), jnp.float32),
          pltpu.VMEM((L,), jnp.float32),
          pltpu.VMEM((L,), jnp.float32),
          pltpu.VMEM((L,), jnp.float32),
          pltpu.VMEM((L,), jnp.float32),
      ],
  )
  out = node_kernel(xf, ex_acc, exh_acc, mvec, wv, av, dv, bv)
  return out.reshape(N, 1)


def kernel(x, edge_index, W, att_src, att_dst, bias):
  return _gat_impl(x, edge_index, W, att_src, att_dst, bias)


# speedup vs baseline: 487.5884x; 1.2728x over previous
import functools

import jax
import jax.numpy as jnp
from jax import lax
from jax.experimental import pallas as pl
from jax.experimental.pallas import tpu as pltpu
from jax.experimental.pallas import tpu_sc as plsc

N = 100000
E = 6400000
ROW = 128
NROWS = E // ROW
NC, NS, L = 2, 16, 16
NW = NC * NS
SCROWS = 16
NPAIR = 48
ZCH = 1000
NZ = N // ZCH
ZCHN = 4000
NZN = N // ZCHN
NEG_SLOPE = 0.2
CLAMP = -75.0


def _edge_body(x_hbm, src_hbm, dst_hbm, c_hbm,
               ex_hbm, exh_hbm, m_hbm,
               hx, srcb0, dstb0, srcb1, dstb1,
               exsb0, exhb0, exsb1, exhb1, zbuf, cbuf,
               acc_ex, acc_exh, seml, sems0, sems1):
  cid = lax.axis_index("c")
  sid = lax.axis_index("s")
  wid = sid * NC + cid

  pltpu.sync_copy(c_hbm, cbuf.at[pl.ds(0, 48)])
  pltpu.sync_copy(x_hbm, hx)
  wv = cbuf[pl.ds(0, L)]
  av = cbuf[pl.ds(L, L)]
  dv = cbuf[pl.ds(2 * L, L)]

  ninf = jnp.full((L,), -jnp.inf, jnp.float32)

  def hbody(j, carry):
    mas, mad = carry
    sl = pl.ds(j * L, L)
    v = hx[sl] * wv
    hx[sl] = v
    return jnp.maximum(mas, v * av), jnp.maximum(mad, v * dv)

  mas, mad = lax.fori_loop(0, N // L, hbody, (ninf, ninf))
  s_as = mas[0]
  s_ad = mad[0]
  for i in range(1, L):
    s_as = jnp.maximum(s_as, mas[i])
    s_ad = jnp.maximum(s_ad, mad[i])
  msum = s_as + s_ad
  mglob = jnp.maximum(msum, NEG_SLOPE * msum)
  mv = jnp.full((L,), mglob, jnp.float32)

  zv = jnp.zeros((L,), jnp.float32)

  def zbody(j, _):
    zbuf[pl.ds(j * L, L)] = zv
    return 0

  lax.fori_loop(0, ZCH // L, zbody, 0)
  for t in range((NZ + NS - 1) // NS):
    k = sid + t * NS

    @pl.when(k < NZ)
    def _():
      pltpu.sync_copy(zbuf, acc_ex.at[pl.ds(k * ZCH, ZCH)])
      pltpu.sync_copy(zbuf, acc_exh.at[pl.ds(k * ZCH, ZCH)])

  plsc.subcore_barrier()

  base = 1560 * wid + 8 * jnp.minimum(wid, 10)
  nrem8 = jnp.where(wid < 10, 4, 3)

  def compute_chunk(srcb, dstb, exsb, exhb, nrows):
    def rbody(j, _):
      for k in range(ROW // L):
        sl = pl.ds(k * L, L)
        sv = srcb[j, sl]
        dvec = dstb[j, sl]
        hs = plsc.load_gather(hx, [sv])
        hd = plsc.load_gather(hx, [dvec])
        e = av * hs + dv * hd
        e = jnp.maximum(e, NEG_SLOPE * e)
        ex = jnp.exp(jnp.maximum(e - mv, CLAMP))
        exsb[j, sl] = ex
        exhb[j, sl] = ex * hs
      return 0

    lax.fori_loop(0, nrows, rbody, 0)

  def fire_load(r0, srcb, dstb):
    pltpu.async_copy(src_hbm.at[pl.ds(r0, SCROWS)], srcb, seml)
    pltpu.async_copy(dst_hbm.at[pl.ds(r0, SCROWS)], dstb, seml)

  def wait_load(r0, srcb, dstb):
    pltpu.make_async_copy(src_hbm.at[pl.ds(r0, SCROWS)], srcb, seml).wait()
    pltpu.make_async_copy(dst_hbm.at[pl.ds(r0, SCROWS)], dstb, seml).wait()

  def fire_scatter(exsb, exhb, dstb, sem, nrows):
    for j in range(nrows):
      pltpu.async_copy(exsb.at[j], acc_ex.at[dstb.at[j]], sem, add=True)
      pltpu.async_copy(exhb.at[j], acc_exh.at[dstb.at[j]], sem, add=True)

  def drain_scatter(exsb, exhb, dstb, sem, nrows):
    for j in range(nrows):
      pltpu.make_async_copy(exsb.at[j], acc_ex.at[dstb.at[j]], sem).wait()
      pltpu.make_async_copy(exhb.at[j], acc_exh.at[dstb.at[j]], sem).wait()

  fire_load(base, srcb0, dstb0)

  def pairbody(s2, _):
    r0 = base + s2 * (2 * SCROWS)
    r1 = r0 + SCROWS
    wait_load(r0, srcb0, dstb0)
    fire_load(r1, srcb1, dstb1)
    compute_chunk(srcb0, dstb0, exsb0, exhb0, SCROWS)
    wait_load(r1, srcb1, dstb1)
    cps0 = []
    for j in range(SCROWS):
      cps0.append(pltpu.async_copy(
          exsb0.at[j], acc_ex.at[dstb0.at[j]], sems0, add=True))
      cps0.append(pltpu.async_copy(
          exhb0.at[j], acc_exh.at[dstb0.at[j]], sems0, add=True))
    compute_chunk(srcb1, dstb1, exsb1, exhb1, SCROWS)
    for cp in cps0:
      cp.wait()
    cps1 = []
    for j in range(SCROWS):
      cps1.append(pltpu.async_copy(
          exsb1.at[j], acc_ex.at[dstb1.at[j]], sems1, add=True))
      cps1.append(pltpu.async_copy(
          exhb1.at[j], acc_exh.at[dstb1.at[j]], sems1, add=True))
    for cp in cps1:
      cp.wait()
    fire_load(r1 + SCROWS, srcb0, dstb0)
    return 0

  lax.fori_loop(0, NPAIR, pairbody, 0)

  rbase = base + NPAIR * 2 * SCROWS
  wait_load(rbase, srcb0, dstb0)

  def rembody(r, _):
    r0 = rbase + r * 8
    pltpu.sync_copy(src_hbm.at[pl.ds(r0, 8)], srcb0.at[pl.ds(0, 8)])
    pltpu.sync_copy(dst_hbm.at[pl.ds(r0, 8)], dstb0.at[pl.ds(0, 8)])
    compute_chunk(srcb0, dstb0, exsb0, exhb0, 8)
    fire_scatter(exsb0, exhb0, dstb0, sems0, 8)
    drain_scatter(exsb0, exhb0, dstb0, sems0, 8)
    return 0

  lax.fori_loop(0, nrem8, rembody, 0)

  plsc.subcore_barrier()

  for t in range((NZ + NS - 1) // NS):
    k = sid + t * NS

    @pl.when(k < NZ)
    def _():
      off = k * ZCH
      pltpu.sync_copy(acc_ex.at[pl.ds(off, ZCH)], zbuf)
      pltpu.sync_copy(zbuf, ex_hbm.at[pl.ds(cid * N + off, ZCH)])
      pltpu.sync_copy(acc_exh.at[pl.ds(off, ZCH)], zbuf)
      pltpu.sync_copy(zbuf, exh_hbm.at[pl.ds(cid * N + off, ZCH)])

  @pl.when(jnp.logical_and(cid == 0, sid == 0))
  def _():
    cbuf[pl.ds(48, L)] = mv
    pltpu.sync_copy(cbuf.at[pl.ds(48, L)], m_hbm)


def _node_body(x_hbm, ex_hbm, exh_hbm, m_hbm, w_hbm, a_hbm, d_hbm, b_hbm,
               out_hbm,
               xb, e0b, e1b, h0b, h1b, ob, wb, ab, db, bb, mb):
  cid = lax.axis_index("c")
  sid = lax.axis_index("s")
  wid = sid * NC + cid

  pltpu.sync_copy(w_hbm, wb)
  pltpu.sync_copy(a_hbm, ab)
  pltpu.sync_copy(d_hbm, db)
  pltpu.sync_copy(b_hbm, bb)
  pltpu.sync_copy(m_hbm, mb)

  @pl.when(wid < NZN)
  def _():
    off = wid * ZCHN
    sl_h = pl.ds(off, ZCHN)
    pltpu.sync_copy(x_hbm.at[sl_h], xb)
    pltpu.sync_copy(ex_hbm.at[pl.ds(off, ZCHN)], e0b)
    pltpu.sync_copy(ex_hbm.at[pl.ds(N + off, ZCHN)], e1b)
    pltpu.sync_copy(exh_hbm.at[pl.ds(off, ZCHN)], h0b)
    pltpu.sync_copy(exh_hbm.at[pl.ds(N + off, ZCHN)], h1b)
    wv = wb[...]
    av = ab[...]
    dv = db[...]
    bv = bb[...]
    mv = mb[...]

    def nbody(j, _):
      sl = pl.ds(j * L, L)
      h = xb[sl] * wv
      t = (av + dv) * h
      t = jnp.maximum(t, NEG_SLOPE * t)
      es = jnp.exp(jnp.maximum(t - mv, CLAMP))
      num = h0b[sl] + h1b[sl] + es * h
      den = e0b[sl] + e1b[sl] + es
      ob[sl] = num / den + bv
      return 0

    lax.fori_loop(0, ZCHN // L, nbody, 0)
    pltpu.sync_copy(ob, out_hbm.at[sl_h])


def _gat_impl(x, edge_index, W, att_src, att_dst, bias):
  src = edge_index[0].astype(jnp.int32).reshape(NROWS, ROW)
  dst = edge_index[1].astype(jnp.int32).reshape(NROWS, ROW)
  xf = x.reshape(N).astype(jnp.float32)
  wv = jnp.broadcast_to(W.reshape(()).astype(jnp.float32), (L,))
  av = jnp.broadcast_to(att_src.reshape(()).astype(jnp.float32), (L,))
  dv = jnp.broadcast_to(att_dst.reshape(()).astype(jnp.float32), (L,))
  bv = jnp.broadcast_to(bias.reshape(()).astype(jnp.float32), (L,))
  cin = jnp.concatenate([wv, av, dv])

  mesh = plsc.VectorSubcoreMesh(
      core_axis_name="c", subcore_axis_name="s",
      num_cores=NC, num_subcores=NS)

  cparams = pltpu.CompilerParams(needs_layout_passes=False)
  edge_kernel = pl.kernel(
      _edge_body,
      compiler_params=cparams,
      out_type=(
          jax.ShapeDtypeStruct((NC * N,), jnp.float32),
          jax.ShapeDtypeStruct((NC * N,), jnp.float32),
          jax.ShapeDtypeStruct((L,), jnp.float32),
      ),
      mesh=mesh,
      scratch_types=[
          pltpu.VMEM((N,), jnp.float32),
          pltpu.VMEM((SCROWS, ROW), jnp.int32),
          pltpu.VMEM((SCROWS, ROW), jnp.int32),
          pltpu.VMEM((SCROWS, ROW), jnp.int32),
          pltpu.VMEM((SCROWS, ROW), jnp.int32),
          pltpu.VMEM((SCROWS, ROW), jnp.float32),
          pltpu.VMEM((SCROWS, ROW), jnp.float32),
          pltpu.VMEM((SCROWS, ROW), jnp.float32),
          pltpu.VMEM((SCROWS, ROW), jnp.float32),
          pltpu.VMEM((ZCH,), jnp.float32),
          pltpu.VMEM((64,), jnp.float32),
          pltpu.VMEM_SHARED((N,), jnp.float32),
          pltpu.VMEM_SHARED((N,), jnp.float32),
          pltpu.SemaphoreType.DMA,
          pltpu.SemaphoreType.DMA,
          pltpu.SemaphoreType.DMA,
      ],
  )
  ex_acc, exh_acc, mvec = edge_kernel(xf, src, dst, cin)

  node_kernel = pl.kernel(
      _node_body,
      compiler_params=cparams,
      out_type=jax.ShapeDtypeStruct((N,), jnp.float32),
      mesh=mesh,
      scratch_types=[
          pltpu.VMEM((ZCHN,), jnp.float32),
          pltpu.VMEM((ZCHN,), jnp.float32),
          pltpu.VMEM((ZCHN,), jnp.float32),
          pltpu.VMEM((ZCHN,), jnp.float32),
          pltpu.VMEM((ZCHN,), jnp.float32),
          pltpu.VMEM((ZCHN,), jnp.float32),
          pltpu.VMEM((L,), jnp.float32),
          pltpu.VMEM((L,), jnp.float32),
          pltpu.VMEM((L,), jnp.float32),
          pltpu.VMEM((L,), jnp.float32),
          pltpu.VMEM((L,), jnp.float32),
      ],
  )
  out = node_kernel(xf, ex_acc, exh_acc, mvec, wv, av, dv, bv)
  return out.reshape(N, 1)


def kernel(x, edge_index, W, att_src, att_dst, bias):
  return _gat_impl(x, edge_index, W, att_src, att_dst, bias)
